# Initial kernel scaffold; baseline (speedup 1.0000x reference)
import jax
import jax.numpy as jnp
from jax.experimental import pallas as pl


def kernel(params, x_cat, edge_index, edge_attr, batch_vec):
    raise NotImplementedError("write your pallas kernel here")


# speedup vs baseline: 1.0622x; 1.0622x over previous
import functools

import jax
import jax.numpy as jnp
from jax import lax
from jax.experimental import pallas as pl
from jax.experimental.pallas import tpu as pltpu
from jax.experimental.pallas import tpu_sc as plsc

H = 128
F32 = jnp.float32
I32 = jnp.int32

NW = 32
CE = 128
CH = 160
EP = 16 * CH * CE
HH = H // 2
NREAL = 10000
NP = 10240
GR = 64
ETAB = 322

_MESH = plsc.VectorSubcoreMesh(core_axis_name="c", subcore_axis_name="s")


def _mm(a, b):
    return lax.dot_general(a, b, (((1,), (0,)), ((), ())),
                           precision=lax.Precision.HIGHEST,
                           preferred_element_type=F32)


def _ln(x, g, b):
    mu = jnp.mean(x, axis=-1, keepdims=True)
    v = jnp.mean((x - mu) ** 2, axis=-1, keepdims=True)
    return (x - mu) / jnp.sqrt(v + 1e-5) * g + b


def _gelu(x):
    return 0.5 * x * (1.0 + lax.erf(x * 0.7071067811865476))


def _h0_body(atab, acode3, out, idx_v, rows_v, sem):
    wid = lax.axis_index("s") * 2 + lax.axis_index("c")
    pltpu.sync_copy(acode3.at[wid], idx_v)

    def step(j, carry):
        pltpu.async_copy(atab.at[idx_v.at[j]], rows_v, sem).wait()
        pltpu.sync_copy(rows_v, out.at[pl.ds(wid * 320 + j * 80, 80)])
        return carry

    lax.fori_loop(0, 4, step, 0)


def _agg_body(h2, eltab2, src3, ec3, dst3, zeros, out,
              srcv, ecv, dstv, hrows, elrows, aggr_sh, sem):
    cid = lax.axis_index("c")
    sid = lax.axis_index("s")
    wid = cid * 16 + sid
    rows_per_tile = NP // 16
    r0 = sid * rows_per_tile
    pltpu.sync_copy(zeros.at[pl.ds(r0, rows_per_tile)],
                    aggr_sh.at[pl.ds(r0, rows_per_tile)])
    pltpu.sync_copy(src3.at[wid], srcv)
    pltpu.sync_copy(ec3.at[wid], ecv)
    pltpu.sync_copy(dst3.at[sid], dstv)
    plsc.subcore_barrier()

    def chunk(j, carry):
        c1 = pltpu.async_copy(h2.at[srcv.at[j]], hrows, sem)
        c2 = pltpu.async_copy(eltab2.at[ecv.at[j]], elrows, sem)
        c1.wait()
        c2.wait()

        def row(r, carry2):
            for q in range(HH // 16):
                s = pl.ds(q * 16, 16)
                hrows[r, s] = jnp.maximum(hrows[r, s] + elrows[r, s], 0.0)
            return carry2

        lax.fori_loop(0, CE, row, 0)
        pltpu.sync_copy(hrows, aggr_sh.at[dstv.at[j]], add=True)
        return carry

    lax.fori_loop(0, CH, chunk, 0)
    plsc.subcore_barrier()
    pltpu.sync_copy(aggr_sh.at[pl.ds(r0, rows_per_tile)],
                    out.at[pl.ds(cid * NP + r0, rows_per_tile)])


def _node_body(h, p, w1, b1, w2, b2, g, b, out):
    aggr = jnp.concatenate([p[0:NP, :], p[NP:2 * NP, :]], axis=-1)
    z = h[...] + aggr
    t = _gelu(_mm(z, w1[...]) + b1[...])
    z2 = _mm(t, w2[...]) + b2[...]
    out[...] = _gelu(_ln(z2, g[...], b[...]))


def _pool_body(h, bv, plng, plnb, pw, pb, p1w, p1b, plg, plb,
               rlg, rlb, rw1, rb1, rw2, rb2, p2w, p2b, out, hmax_s):
    hh = h[0:NREAL, :]
    ids = bv[...]
    onehot = (ids == lax.broadcasted_iota(I32, (1, GR), 1)).astype(F32)
    hadd = lax.dot_general(onehot, hh, (((0,), (0,)), ((), ())),
                           precision=lax.Precision.HIGHEST,
                           preferred_element_type=F32)
    counts = lax.dot_general(onehot, jnp.ones((NREAL, 1), F32),
                             (((0,), (0,)), ((), ())),
                             precision=lax.Precision.HIGHEST,
                             preferred_element_type=F32)
    hmean = hadd / jnp.maximum(counts, 1.0)

    def step(gi, carry):
        wh = jnp.where(ids == gi, hh, -1e30)
        hmax_s[pl.ds(gi, 1), :] = jnp.max(wh, axis=0, keepdims=True)
        return carry

    lax.fori_loop(0, GR, step, 0)
    hmax = jnp.where(counts > 0.0, hmax_s[...], 0.0)

    gc = jnp.concatenate([hmean, hmax, hadd], axis=-1)
    gc = _gelu(_mm(_ln(gc, plng[...], plnb[...]), pw[...]) + pb[...])
    y = _mm(gc, p1w[...]) + p1b[...]
    y = _gelu(_ln(y, plg[...], plb[...]))
    r = _ln(y, rlg[...], rlb[...])
    r = _mm(_gelu(_mm(r, rw1[...]) + rb1[...]), rw2[...]) + rb2[...]
    y = y + r
    o = _mm(y, p2w[...]) + p2b[...]
    nrm = jnp.sqrt(jnp.sum(o * o, axis=-1, keepdims=True))
    out[...] = o / jnp.maximum(nrm, 1e-12)


def kernel(params, x_cat, edge_index, edge_attr, batch_vec):
    A = [params["atom_emb"][i] @ params["atom_proj_w"][i * H:(i + 1) * H]
         for i in range(9)]
    base = sum(a[0] for a in A) + params["atom_proj_b"]
    delta = jnp.stack([a[1] - a[0] for a in A])
    bits = ((jnp.arange(512)[:, None] >> jnp.arange(9)[None, :]) & 1)
    atab = base[None, :] + bits.astype(F32) @ delta

    B = [params["bond_emb"][i] @ params["bond_proj_w"][i * H:(i + 1) * H]
         for i in range(3)]
    efull = (B[0][:, None, None, :] + B[1][None, :, None, :]
             + B[2][None, None, :, :]
             + params["bond_proj_b"]).reshape(ETAB, H)
    eltabs = [efull @ cp["lin_w"] + cp["lin_b"] for cp in params["convs"]]

    acode = (x_cat * (1 << jnp.arange(9, dtype=I32))[None, :]).sum(
        axis=1, dtype=I32)
    acode3 = jnp.concatenate(
        [acode, jnp.zeros((NP - NREAL,), I32)]).reshape(NW, 4, 80)
    ecode = ((edge_attr[:, 0] * 7 + edge_attr[:, 1]) * 2
             + edge_attr[:, 2]).astype(I32)
    padn = EP - ecode.shape[0]
    spread = jnp.arange(padn, dtype=I32) % 64
    src1 = jnp.concatenate([edge_index[0], spread]).reshape(16, CH, CE)
    ec1 = jnp.concatenate([ecode, jnp.zeros((padn,), I32)]).reshape(16, CH, CE)
    src3 = jnp.concatenate([src1, src1 + NP])
    ec3 = jnp.concatenate([ec1, ec1 + ETAB])
    dst3 = jnp.concatenate([edge_index[1], NREAL + spread]).reshape(16, CH, CE)
    zeros = jnp.zeros((NP, HH), F32)

    h = pl.kernel(
        _h0_body,
        out_type=jax.ShapeDtypeStruct((NP, H), F32),
        mesh=_MESH,
        scratch_types=[pltpu.VMEM((4, 80), I32),
                       pltpu.VMEM((80, H), F32),
                       pltpu.SemaphoreType.DMA],
    )(atab, acode3)

    agg_call = pl.kernel(
        _agg_body,
        out_type=jax.ShapeDtypeStruct((2 * NP, HH), F32),
        mesh=_MESH,
        scratch_types=[pltpu.VMEM((CH, CE), I32),
                       pltpu.VMEM((CH, CE), I32),
                       pltpu.VMEM((CH, CE), I32),
                       pltpu.VMEM((CE, HH), F32),
                       pltpu.VMEM((CE, HH), F32),
                       pltpu.VMEM_SHARED((NP, HH), F32),
                       pltpu.SemaphoreType.DMA],
        compiler_params=pltpu.CompilerParams(use_tc_tiling_on_sc=False),
    )

    node_call = pl.pallas_call(
        _node_body, out_shape=jax.ShapeDtypeStruct((NP, H), F32))

    r1 = lambda a: a.reshape(1, -1)
    for l, (cp, npm) in enumerate(zip(params["convs"], params["norms"])):
        h2 = jnp.concatenate([h[:, :HH], h[:, HH:]])
        el2 = jnp.concatenate([eltabs[l][:, :HH], eltabs[l][:, HH:]])
        p = agg_call(h2, el2, src3, ec3, dst3, zeros)
        h = node_call(h, p, cp["w1"], r1(cp["b1"]), cp["w2"], r1(cp["b2"]),
                      r1(npm["g"]), r1(npm["b"]))

    out = pl.pallas_call(
        _pool_body,
        out_shape=jax.ShapeDtypeStruct((GR, 768), F32),
        scratch_shapes=[pltpu.VMEM((GR, H), F32)],
    )(h, batch_vec.reshape(NREAL, 1),
      r1(params["pool_ln_g"]), r1(params["pool_ln_b"]),
      params["pool_w"], r1(params["pool_b"]),
      params["p1_w"], r1(params["p1_b"]),
      r1(params["p_ln_g"]), r1(params["p_ln_b"]),
      r1(params["r_ln_g"]), r1(params["r_ln_b"]),
      params["r_w1"], r1(params["r_b1"]),
      params["r_w2"], r1(params["r_b2"]),
      params["p2_w"], r1(params["p2_b"]))
    return out


# speedup vs baseline: 4.2049x; 3.9588x over previous
import functools

import jax
import jax.numpy as jnp
from jax import lax
from jax.experimental import pallas as pl
from jax.experimental.pallas import tpu as pltpu
from jax.experimental.pallas import tpu_sc as plsc

H = 128
F32 = jnp.float32
I32 = jnp.int32

NW = 32
CE = 128
CH = 160
EP = 16 * CH * CE
EP2 = 16 * (CH + 1) * CE
HH = H // 2
REP = 16
NREAL = 10000
NP = 10240
GR = 64
ETAB = 322

_MESH = plsc.VectorSubcoreMesh(core_axis_name="c", subcore_axis_name="s")


def _mm(a, b):
    return lax.dot_general(a, b, (((1,), (0,)), ((), ())),
                           precision=lax.Precision.HIGHEST,
                           preferred_element_type=F32)


def _ln(x, g, b):
    mu = jnp.mean(x, axis=-1, keepdims=True)
    v = jnp.mean((x - mu) ** 2, axis=-1, keepdims=True)
    return (x - mu) / jnp.sqrt(v + 1e-5) * g + b


def _gelu(x):
    return 0.5 * x * (1.0 + lax.erf(x * 0.7071067811865476))


def _h0_body(atab, acode3, out, idx_v, rows_v, sem):
    wid = lax.axis_index("s") * 2 + lax.axis_index("c")
    pltpu.sync_copy(acode3.at[wid], idx_v)

    def step(j, carry):
        pltpu.async_copy(atab.at[idx_v.at[j]], rows_v, sem).wait()
        pltpu.sync_copy(rows_v, out.at[pl.ds(wid * 320 + j * 80, 80)])
        return carry

    lax.fori_loop(0, 4, step, 0)


def _agg_body(h2, eltab2, comb3, dst3, out,
              combv, dstv, srcx, ecx, hrows, hrows2, elrows, elrows2,
              aggr_sh, sem, sem2):
    cid = lax.axis_index("c")
    sid = lax.axis_index("s")
    wid = cid * 16 + sid
    rows_per_tile = NP // 16
    r0 = sid * rows_per_tile
    pltpu.sync_copy(comb3.at[wid], combv)
    pltpu.sync_copy(dst3.at[sid], dstv)

    def zrow(r, c):
        for q in range(HH // 16):
            hrows[r, pl.ds(q * 16, 16)] = jnp.zeros((16,), F32)
        return c

    lax.fori_loop(0, CE, zrow, 0)
    for k in range(rows_per_tile // CE):
        pltpu.sync_copy(hrows, aggr_sh.at[pl.ds(r0 + k * CE, CE)])
    plsc.subcore_barrier()

    hb = [hrows, hrows2]
    eb = [elrows, elrows2]
    sems = [sem, sem2]

    def fire(j, b):
        for q in range(CE // 16):
            s = pl.ds(q * 16, 16)
            v = combv[j, s]
            srcx[b, s] = jnp.bitwise_and(v, 32767)
            ecx[b, s] = lax.shift_right_logical(v, 15)
        pltpu.async_copy(h2.at[srcx.at[b]], hb[b], sems[b])
        pltpu.async_copy(eltab2.at[ecx.at[b]], eb[b], sems[b])

    def drain(j, b):
        pltpu.make_async_copy(h2.at[srcx.at[b]], hb[b], sems[b]).wait()
        pltpu.make_async_copy(eltab2.at[ecx.at[b]], eb[b], sems[b]).wait()

    def work(j, b):
        drain(j, b)
        hr, er = hb[b], eb[b]

        def row(r2, carry2):
            r = r2 * 2
            for q in range(2 * HH // 16):
                rr = r + q // (HH // 16)
                s = pl.ds((q % (HH // 16)) * 16, 16)
                hr[rr, s] = jnp.maximum(hr[rr, s] + er[rr, s], 0.0)
            return carry2

        lax.fori_loop(0, CE // 2, row, 0)
        pltpu.sync_copy(hr, aggr_sh.at[dstv.at[j]], add=True)

    fire(0, 0)

    def pair(jj, carry):
        j0 = 2 * jj
        fire(j0 + 1, 1)
        work(j0, 0)
        fire(j0 + 2, 0)
        work(j0 + 1, 1)
        return carry

    lax.fori_loop(0, CH // 2, pair, 0)
    drain(CH, 0)
    plsc.subcore_barrier()
    pltpu.sync_copy(aggr_sh.at[pl.ds(r0, rows_per_tile)],
                    out.at[pl.ds(cid * NP + r0, rows_per_tile)])


def _node_body(h, p, w1, b1, w2, b2, g, b, out):
    aggr = jnp.concatenate([p[0:NP, :], p[NP:2 * NP, :]], axis=-1)
    z = h[...] + aggr
    t = _gelu(_mm(z, w1[...]) + b1[...])
    z2 = _mm(t, w2[...]) + b2[...]
    out[...] = _gelu(_ln(z2, g[...], b[...]))


def _pool_body(h, bv, plng, plnb, pw, pb, p1w, p1b, plg, plb,
               rlg, rlb, rw1, rb1, rw2, rb2, p2w, p2b, out, hmax_s):
    hh = h[0:NREAL, :]
    ids = bv[...]
    onehot = (ids == lax.broadcasted_iota(I32, (1, GR), 1)).astype(F32)
    hadd = lax.dot_general(onehot, hh, (((0,), (0,)), ((), ())),
                           precision=lax.Precision.HIGHEST,
                           preferred_element_type=F32)
    counts = lax.dot_general(onehot, jnp.ones((NREAL, 1), F32),
                             (((0,), (0,)), ((), ())),
                             precision=lax.Precision.HIGHEST,
                             preferred_element_type=F32)
    hmean = hadd / jnp.maximum(counts, 1.0)

    def step(gi, carry):
        wh = jnp.where(ids == gi, hh, -1e30)
        hmax_s[pl.ds(gi, 1), :] = jnp.max(wh, axis=0, keepdims=True)
        return carry

    lax.fori_loop(0, GR, step, 0)
    hmax = jnp.where(counts > 0.0, hmax_s[...], 0.0)

    gc = jnp.concatenate([hmean, hmax, hadd], axis=-1)
    gc = _gelu(_mm(_ln(gc, plng[...], plnb[...]), pw[...]) + pb[...])
    y = _mm(gc, p1w[...]) + p1b[...]
    y = _gelu(_ln(y, plg[...], plb[...]))
    r = _ln(y, rlg[...], rlb[...])
    r = _mm(_gelu(_mm(r, rw1[...]) + rb1[...]), rw2[...]) + rb2[...]
    y = y + r
    o = _mm(y, p2w[...]) + p2b[...]
    nrm = jnp.sqrt(jnp.sum(o * o, axis=-1, keepdims=True))
    out[...] = o / jnp.maximum(nrm, 1e-12)


def kernel(params, x_cat, edge_index, edge_attr, batch_vec):
    A = [params["atom_emb"][i] @ params["atom_proj_w"][i * H:(i + 1) * H]
         for i in range(9)]
    base = sum(a[0] for a in A) + params["atom_proj_b"]
    delta = jnp.stack([a[1] - a[0] for a in A])
    bits = ((jnp.arange(512)[:, None] >> jnp.arange(9)[None, :]) & 1)
    atab = base[None, :] + bits.astype(F32) @ delta

    B = [params["bond_emb"][i] @ params["bond_proj_w"][i * H:(i + 1) * H]
         for i in range(3)]
    efull = (B[0][:, None, None, :] + B[1][None, :, None, :]
             + B[2][None, None, :, :]
             + params["bond_proj_b"]).reshape(ETAB, H)
    eltabs = [efull @ cp["lin_w"] + cp["lin_b"] for cp in params["convs"]]

    acode = (x_cat * (1 << jnp.arange(9, dtype=I32))[None, :]).sum(
        axis=1, dtype=I32)
    acode3 = jnp.concatenate(
        [acode, jnp.zeros((NP - NREAL,), I32)]).reshape(NW, 4, 80)
    ecode = ((edge_attr[:, 0] * 7 + edge_attr[:, 1]) * 2
             + edge_attr[:, 2]).astype(I32)
    padn = EP - ecode.shape[0]
    spread = jnp.arange(padn, dtype=I32) % 64
    padchunk = (jnp.arange(16 * CE, dtype=I32) % 64).reshape(16, 1, CE)
    src1 = jnp.concatenate(
        [jnp.concatenate([edge_index[0], spread]).reshape(16, CH, CE),
         padchunk], axis=1)
    rep_off = (jnp.arange(EP, dtype=I32) % REP) * ETAB
    ec1 = jnp.concatenate(
        [(jnp.concatenate([ecode, jnp.zeros((padn,), I32)])
          + rep_off).reshape(16, CH, CE),
         jnp.zeros((16, 1, CE), I32)], axis=1)
    src3 = jnp.concatenate([src1, src1 + NP])
    ec3 = jnp.concatenate([ec1, ec1 + REP * ETAB])
    comb3 = ec3 * 32768 + src3
    dst3 = jnp.concatenate([edge_index[1], NREAL + spread]).reshape(16, CH, CE)

    h = pl.kernel(
        _h0_body,
        out_type=jax.ShapeDtypeStruct((NP, H), F32),
        mesh=_MESH,
        scratch_types=[pltpu.VMEM((4, 80), I32),
                       pltpu.VMEM((80, H), F32),
                       pltpu.SemaphoreType.DMA],
    )(atab, acode3)

    agg_call = pl.kernel(
        _agg_body,
        out_type=jax.ShapeDtypeStruct((2 * NP, HH), F32),
        mesh=_MESH,
        scratch_types=[pltpu.VMEM((CH + 1, CE), I32),
                       pltpu.VMEM((CH, CE), I32),
                       pltpu.VMEM((2, CE), I32),
                       pltpu.VMEM((2, CE), I32),
                       pltpu.VMEM((CE, HH), F32),
                       pltpu.VMEM((CE, HH), F32),
                       pltpu.VMEM((CE, HH), F32),
                       pltpu.VMEM((CE, HH), F32),
                       pltpu.VMEM_SHARED((NP, HH), F32),
                       pltpu.SemaphoreType.DMA,
                       pltpu.SemaphoreType.DMA],
        compiler_params=pltpu.CompilerParams(use_tc_tiling_on_sc=False),
    )

    node_call = pl.pallas_call(
        _node_body, out_shape=jax.ShapeDtypeStruct((NP, H), F32))

    r1 = lambda a: a.reshape(1, -1)
    for l, (cp, npm) in enumerate(zip(params["convs"], params["norms"])):
        h2 = jnp.concatenate([h[:, :HH], h[:, HH:]])
        el2 = jnp.concatenate([jnp.tile(eltabs[l][:, :HH], (REP, 1)),
                               jnp.tile(eltabs[l][:, HH:], (REP, 1))])
        p = agg_call(h2, el2, comb3, dst3)
        h = node_call(h, p, cp["w1"], r1(cp["b1"]), cp["w2"], r1(cp["b2"]),
                      r1(npm["g"]), r1(npm["b"]))

    out = pl.pallas_call(
        _pool_body,
        out_shape=jax.ShapeDtypeStruct((GR, 768), F32),
        scratch_shapes=[pltpu.VMEM((GR, H), F32)],
    )(h, batch_vec.reshape(NREAL, 1),
      r1(params["pool_ln_g"]), r1(params["pool_ln_b"]),
      params["pool_w"], r1(params["pool_b"]),
      params["p1_w"], r1(params["p1_b"]),
      r1(params["p_ln_g"]), r1(params["p_ln_b"]),
      r1(params["r_ln_g"]), r1(params["r_ln_b"]),
      params["r_w1"], r1(params["r_b1"]),
      params["r_w2"], r1(params["r_b2"]),
      params["p2_w"], r1(params["p2_b"]))
    return out


# speedup vs baseline: 4.2089x; 1.0010x over previous
import functools

import jax
import jax.numpy as jnp
from jax import lax
from jax.experimental import pallas as pl
from jax.experimental.pallas import tpu as pltpu
from jax.experimental.pallas import tpu_sc as plsc

H = 128
F32 = jnp.float32
I32 = jnp.int32

NW = 32
CE = 128
CH = 160
EP = 16 * CH * CE
EP2 = 16 * (CH + 1) * CE
HH = H // 2
REP = 16
NREAL = 10000
NP = 10240
GR = 64
ETAB = 322

_MESH = plsc.VectorSubcoreMesh(core_axis_name="c", subcore_axis_name="s")


def _mm(a, b):
    return lax.dot_general(a, b, (((1,), (0,)), ((), ())),
                           precision=lax.Precision.HIGHEST,
                           preferred_element_type=F32)


def _ln(x, g, b):
    mu = jnp.mean(x, axis=-1, keepdims=True)
    v = jnp.mean((x - mu) ** 2, axis=-1, keepdims=True)
    return (x - mu) / jnp.sqrt(v + 1e-5) * g + b


def _gelu(x):
    return 0.5 * x * (1.0 + lax.erf(x * 0.7071067811865476))


def _h0_body(atab, acode3, out, idx_v, rows_v, sem):
    wid = lax.axis_index("s") * 2 + lax.axis_index("c")
    pltpu.sync_copy(acode3.at[wid], idx_v)

    def step(j, carry):
        pltpu.async_copy(atab.at[idx_v.at[j]], rows_v, sem).wait()
        pltpu.sync_copy(rows_v, out.at[pl.ds(wid * 320 + j * 80, 80)])
        return carry

    lax.fori_loop(0, 4, step, 0)


def _agg_body(h2, eltab2, comb3, dst3, out,
              combv, dstv, srcx, ecx, hrows, hrows2, elrows, elrows2,
              aggr_sh, sem, sem2):
    cid = lax.axis_index("c")
    sid = lax.axis_index("s")
    wid = cid * 16 + sid
    rows_per_tile = NP // 16
    r0 = sid * rows_per_tile
    pltpu.sync_copy(comb3.at[wid], combv)
    pltpu.sync_copy(dst3.at[sid], dstv)

    def zrow(r, c):
        for q in range(HH // 16):
            hrows[r, pl.ds(q * 16, 16)] = jnp.zeros((16,), F32)
        return c

    lax.fori_loop(0, CE, zrow, 0)
    for k in range(rows_per_tile // CE):
        pltpu.sync_copy(hrows, aggr_sh.at[pl.ds(r0 + k * CE, CE)])
    plsc.subcore_barrier()

    hb = [hrows, hrows2]
    eb = [elrows, elrows2]
    sems = [sem, sem2]

    def fire(j, b):
        for q in range(CE // 16):
            s = pl.ds(q * 16, 16)
            v = combv[j, s]
            srcx[b, s] = jnp.bitwise_and(v, 32767)
            ecx[b, s] = lax.shift_right_logical(v, 15)
        pltpu.async_copy(h2.at[srcx.at[b]], hb[b], sems[b])
        pltpu.async_copy(eltab2.at[ecx.at[b]], eb[b], sems[b])

    def drain(j, b):
        pltpu.make_async_copy(h2.at[srcx.at[b]], hb[b], sems[b]).wait()
        pltpu.make_async_copy(eltab2.at[ecx.at[b]], eb[b], sems[b]).wait()

    def work(j, b):
        drain(j, b)
        hr, er = hb[b], eb[b]

        @plsc.parallel_loop(0, CE // 2, unroll=2)
        def _(r2):
            r = r2 * 2
            for q in range(2 * HH // 16):
                rr = r + q // (HH // 16)
                s = pl.ds((q % (HH // 16)) * 16, 16)
                hr[rr, s] = jnp.maximum(hr[rr, s] + er[rr, s], 0.0)

        pltpu.sync_copy(hr, aggr_sh.at[dstv.at[j]], add=True)

    fire(0, 0)

    def pair(jj, carry):
        j0 = 2 * jj
        fire(j0 + 1, 1)
        work(j0, 0)
        fire(j0 + 2, 0)
        work(j0 + 1, 1)
        return carry

    lax.fori_loop(0, CH // 2, pair, 0)
    drain(CH, 0)
    plsc.subcore_barrier()
    pltpu.sync_copy(aggr_sh.at[pl.ds(r0, rows_per_tile)],
                    out.at[pl.ds(cid * NP + r0, rows_per_tile)])


def _node_body(h, p, w1, b1, w2, b2, g, b, out):
    aggr = jnp.concatenate([p[0:NP, :], p[NP:2 * NP, :]], axis=-1)
    z = h[...] + aggr
    t = _gelu(_mm(z, w1[...]) + b1[...])
    z2 = _mm(t, w2[...]) + b2[...]
    out[...] = _gelu(_ln(z2, g[...], b[...]))


def _pool_body(h, bv, plng, plnb, pw, pb, p1w, p1b, plg, plb,
               rlg, rlb, rw1, rb1, rw2, rb2, p2w, p2b, out, hmax_s):
    hh = h[0:NREAL, :]
    ids = bv[...]
    onehot = (ids == lax.broadcasted_iota(I32, (1, GR), 1)).astype(F32)
    hadd = lax.dot_general(onehot, hh, (((0,), (0,)), ((), ())),
                           precision=lax.Precision.HIGHEST,
                           preferred_element_type=F32)
    counts = lax.dot_general(onehot, jnp.ones((NREAL, 1), F32),
                             (((0,), (0,)), ((), ())),
                             precision=lax.Precision.HIGHEST,
                             preferred_element_type=F32)
    hmean = hadd / jnp.maximum(counts, 1.0)

    def step(gi, carry):
        wh = jnp.where(ids == gi, hh, -1e30)
        hmax_s[pl.ds(gi, 1), :] = jnp.max(wh, axis=0, keepdims=True)
        return carry

    lax.fori_loop(0, GR, step, 0)
    hmax = jnp.where(counts > 0.0, hmax_s[...], 0.0)

    gc = jnp.concatenate([hmean, hmax, hadd], axis=-1)
    gc = _gelu(_mm(_ln(gc, plng[...], plnb[...]), pw[...]) + pb[...])
    y = _mm(gc, p1w[...]) + p1b[...]
    y = _gelu(_ln(y, plg[...], plb[...]))
    r = _ln(y, rlg[...], rlb[...])
    r = _mm(_gelu(_mm(r, rw1[...]) + rb1[...]), rw2[...]) + rb2[...]
    y = y + r
    o = _mm(y, p2w[...]) + p2b[...]
    nrm = jnp.sqrt(jnp.sum(o * o, axis=-1, keepdims=True))
    out[...] = o / jnp.maximum(nrm, 1e-12)


def kernel(params, x_cat, edge_index, edge_attr, batch_vec):
    A = [params["atom_emb"][i] @ params["atom_proj_w"][i * H:(i + 1) * H]
         for i in range(9)]
    base = sum(a[0] for a in A) + params["atom_proj_b"]
    delta = jnp.stack([a[1] - a[0] for a in A])
    bits = ((jnp.arange(512)[:, None] >> jnp.arange(9)[None, :]) & 1)
    atab = base[None, :] + bits.astype(F32) @ delta

    B = [params["bond_emb"][i] @ params["bond_proj_w"][i * H:(i + 1) * H]
         for i in range(3)]
    efull = (B[0][:, None, None, :] + B[1][None, :, None, :]
             + B[2][None, None, :, :]
             + params["bond_proj_b"]).reshape(ETAB, H)
    eltabs = [efull @ cp["lin_w"] + cp["lin_b"] for cp in params["convs"]]

    acode = (x_cat * (1 << jnp.arange(9, dtype=I32))[None, :]).sum(
        axis=1, dtype=I32)
    acode3 = jnp.concatenate(
        [acode, jnp.zeros((NP - NREAL,), I32)]).reshape(NW, 4, 80)
    ecode = ((edge_attr[:, 0] * 7 + edge_attr[:, 1]) * 2
             + edge_attr[:, 2]).astype(I32)
    padn = EP - ecode.shape[0]
    spread = jnp.arange(padn, dtype=I32) % 64
    padchunk = (jnp.arange(16 * CE, dtype=I32) % 64).reshape(16, 1, CE)
    src1 = jnp.concatenate(
        [jnp.concatenate([edge_index[0], spread]).reshape(16, CH, CE),
         padchunk], axis=1)
    rep_off = (jnp.arange(EP, dtype=I32) % REP) * ETAB
    ec1 = jnp.concatenate(
        [(jnp.concatenate([ecode, jnp.zeros((padn,), I32)])
          + rep_off).reshape(16, CH, CE),
         jnp.zeros((16, 1, CE), I32)], axis=1)
    src3 = jnp.concatenate([src1, src1 + NP])
    ec3 = jnp.concatenate([ec1, ec1 + REP * ETAB])
    comb3 = ec3 * 32768 + src3
    dst3 = jnp.concatenate([edge_index[1], NREAL + spread]).reshape(16, CH, CE)

    h = pl.kernel(
        _h0_body,
        out_type=jax.ShapeDtypeStruct((NP, H), F32),
        mesh=_MESH,
        scratch_types=[pltpu.VMEM((4, 80), I32),
                       pltpu.VMEM((80, H), F32),
                       pltpu.SemaphoreType.DMA],
    )(atab, acode3)

    agg_call = pl.kernel(
        _agg_body,
        out_type=jax.ShapeDtypeStruct((2 * NP, HH), F32),
        mesh=_MESH,
        scratch_types=[pltpu.VMEM((CH + 1, CE), I32),
                       pltpu.VMEM((CH, CE), I32),
                       pltpu.VMEM((2, CE), I32),
                       pltpu.VMEM((2, CE), I32),
                       pltpu.VMEM((CE, HH), F32),
                       pltpu.VMEM((CE, HH), F32),
                       pltpu.VMEM((CE, HH), F32),
                       pltpu.VMEM((CE, HH), F32),
                       pltpu.VMEM_SHARED((NP, HH), F32),
                       pltpu.SemaphoreType.DMA,
                       pltpu.SemaphoreType.DMA],
        compiler_params=pltpu.CompilerParams(use_tc_tiling_on_sc=False),
    )

    node_call = pl.pallas_call(
        _node_body, out_shape=jax.ShapeDtypeStruct((NP, H), F32))

    r1 = lambda a: a.reshape(1, -1)
    for l, (cp, npm) in enumerate(zip(params["convs"], params["norms"])):
        h2 = jnp.concatenate([h[:, :HH], h[:, HH:]])
        el2 = jnp.concatenate([jnp.tile(eltabs[l][:, :HH], (REP, 1)),
                               jnp.tile(eltabs[l][:, HH:], (REP, 1))])
        p = agg_call(h2, el2, comb3, dst3)
        h = node_call(h, p, cp["w1"], r1(cp["b1"]), cp["w2"], r1(cp["b2"]),
                      r1(npm["g"]), r1(npm["b"]))

    out = pl.pallas_call(
        _pool_body,
        out_shape=jax.ShapeDtypeStruct((GR, 768), F32),
        scratch_shapes=[pltpu.VMEM((GR, H), F32)],
    )(h, batch_vec.reshape(NREAL, 1),
      r1(params["pool_ln_g"]), r1(params["pool_ln_b"]),
      params["pool_w"], r1(params["pool_b"]),
      params["p1_w"], r1(params["p1_b"]),
      r1(params["p_ln_g"]), r1(params["p_ln_b"]),
      r1(params["r_ln_g"]), r1(params["r_ln_b"]),
      params["r_w1"], r1(params["r_b1"]),
      params["r_w2"], r1(params["r_b2"]),
      params["p2_w"], r1(params["p2_b"]))
    return out
